# baseline (speedup 1.0000x reference)
import jax
import jax.numpy as jnp
from jax import lax
from jax.experimental import pallas as pl
from jax.experimental.pallas import tpu as pltpu
from jax.experimental.pallas import tpu_sc as plsc

D_MODEL = 768
B = 4
S = 4096
EPS = 1e-12
NW = 32
P_PER_W = S // NW
G = 32
NCHUNK = 16
NJ = D_MODEL // 16
TUNROLL = 4


def _ln_chunk(rows_v, pos_v, g_v, b_v):
    def token_body(i, _):
        t0 = i * TUNROLL
        accs = []
        for k in range(TUNROLL):
            t = t0 + k
            acc = jnp.zeros((16,), jnp.float32)
            acc2 = jnp.zeros((16,), jnp.float32)
            for j in range(NJ):
                sl = pl.ds(j * 16, 16)
                x = rows_v[t, sl] + pos_v[t, sl]
                rows_v[t, sl] = x
                acc = acc + x
                acc2 = acc2 + x * x
            accs.append((acc, acc2))
        stats = []
        ones = jnp.ones((16,), jnp.float32)
        for k in range(TUNROLL):
            acc, acc2 = accs[k]
            mean = jnp.sum(acc) * (1.0 / D_MODEL)
            var = jnp.sum(acc2) * (1.0 / D_MODEL) - mean * mean
            mean_v = mean * ones
            v = (var + EPS) * ones
            iv = plsc.bitcast(v, jnp.int32)
            iv = 0x5F3759DF - (iv >> 1)
            y = plsc.bitcast(iv, jnp.float32)
            half_v = 0.5 * v
            for _n in range(3):
                y = y * (1.5 - half_v * y * y)
            stats.append((mean_v, y))
        for j in range(NJ):
            sl = pl.ds(j * 16, 16)
            g = g_v[sl]
            b = b_v[sl]
            for k in range(TUNROLL):
                t = t0 + k
                mean_v, y = stats[k]
                x = rows_v[t, sl]
                rows_v[t, sl] = (x - mean_v) * y * g + b
        return 0

    lax.fori_loop(0, G // TUNROLL, token_body, 0)


def _sc_body(ids_hbm, wt_hbm, pt_hbm, g_hbm, b_hbm, out_hbm,
             idx0, idx1, rows0, rows1, pos_v, g_v, b_v,
             sem_g0, sem_g1, sem_s0, sem_s1):
    wid = lax.axis_index("s") * 2 + lax.axis_index("c")
    p0 = wid * P_PER_W
    pltpu.sync_copy(g_hbm, g_v)
    pltpu.sync_copy(b_hbm, b_v)

    def tok_of(c):
        return (c % B) * S + p0 + (c // B) * G

    def start_gather(c, idx_v, rows_v, sem):
        pltpu.sync_copy(ids_hbm.at[pl.ds(tok_of(c), G)], idx_v)
        pltpu.make_async_copy(wt_hbm.at[idx_v], rows_v, sem).start()

    def wait_gather(idx_v, rows_v, sem):
        pltpu.make_async_copy(wt_hbm.at[idx_v], rows_v, sem).wait()

    def start_store(c, rows_v, sem):
        pltpu.make_async_copy(rows_v, out_hbm.at[pl.ds(tok_of(c), G)], sem).start()

    def wait_store(c, rows_v, sem):
        pltpu.make_async_copy(rows_v, out_hbm.at[pl.ds(tok_of(c), G)], sem).wait()

    start_gather(0, idx0, rows0, sem_g0)

    def pair_body(p, _):
        c0 = 2 * p
        c1 = c0 + 1

        @pl.when(p > 0)
        def _():
            wait_store(c1 - 2, rows1, sem_s1)

        start_gather(c1, idx1, rows1, sem_g1)
        wait_gather(idx0, rows0, sem_g0)

        @pl.when(c0 % B == 0)
        def _():
            pltpu.sync_copy(pt_hbm.at[pl.ds(p0 + (c0 // B) * G, G)], pos_v)

        _ln_chunk(rows0, pos_v, g_v, b_v)
        start_store(c0, rows0, sem_s0)

        @pl.when(c1 + 1 < NCHUNK)
        def _():
            wait_store(c0, rows0, sem_s0)
            start_gather(c1 + 1, idx0, rows0, sem_g0)

        wait_gather(idx1, rows1, sem_g1)
        _ln_chunk(rows1, pos_v, g_v, b_v)
        start_store(c1, rows1, sem_s1)
        return 0

    lax.fori_loop(0, NCHUNK // 2, pair_body, 0)
    wait_store(NCHUNK - 2, rows0, sem_s0)
    wait_store(NCHUNK - 1, rows1, sem_s1)


@jax.jit
def _run(ids_flat, word_table, pos_table, gamma, beta):
    mesh = plsc.VectorSubcoreMesh(core_axis_name="c", subcore_axis_name="s")
    k = pl.kernel(
        _sc_body,
        out_type=jax.ShapeDtypeStruct((B * S, D_MODEL), jnp.float32),
        mesh=mesh,
        compiler_params=pltpu.CompilerParams(needs_layout_passes=False),
        scratch_types=[
            pltpu.VMEM((G,), jnp.int32),
            pltpu.VMEM((G,), jnp.int32),
            pltpu.VMEM((G, D_MODEL), jnp.float32),
            pltpu.VMEM((G, D_MODEL), jnp.float32),
            pltpu.VMEM((G, D_MODEL), jnp.float32),
            pltpu.VMEM((D_MODEL,), jnp.float32),
            pltpu.VMEM((D_MODEL,), jnp.float32),
            pltpu.SemaphoreType.DMA,
            pltpu.SemaphoreType.DMA,
            pltpu.SemaphoreType.DMA,
            pltpu.SemaphoreType.DMA,
        ],
    )
    return k(ids_flat, word_table, pos_table, gamma, beta)


def kernel(input_ids, word_table, pos_table, gamma, beta):
    ids_flat = jnp.reshape(input_ids.astype(jnp.int32), (B * S,))
    out = _run(ids_flat, word_table, pos_table, gamma, beta)
    return jnp.reshape(out, (B, S, D_MODEL))


# speedup vs baseline: 1.1794x; 1.1794x over previous
import jax
import jax.numpy as jnp
from jax import lax
from jax.experimental import pallas as pl
from jax.experimental.pallas import tpu as pltpu
from jax.experimental.pallas import tpu_sc as plsc

D_MODEL = 768
B = 4
S = 4096
EPS = 1e-12
NW = 32
P_PER_W = S // NW
G = 64
NJ = D_MODEL // 16
TUNROLL = 4


def _ln_chunk(rows_v, pos_v, g_v, b_v):
    def token_body(i, _):
        t0 = i * TUNROLL
        accs = []
        for k in range(TUNROLL):
            t = t0 + k
            acc = jnp.zeros((16,), jnp.float32)
            acc2 = jnp.zeros((16,), jnp.float32)
            for j in range(NJ):
                sl = pl.ds(j * 16, 16)
                x = rows_v[t, sl] + pos_v[t, sl]
                rows_v[t, sl] = x
                acc = acc + x
                acc2 = acc2 + x * x
            accs.append((acc, acc2))
        stats = []
        ones = jnp.ones((16,), jnp.float32)
        for k in range(TUNROLL):
            acc, acc2 = accs[k]
            mean = jnp.sum(acc) * (1.0 / D_MODEL)
            var = jnp.sum(acc2) * (1.0 / D_MODEL) - mean * mean
            mean_v = mean * ones
            v = (var + EPS) * ones
            iv = plsc.bitcast(v, jnp.int32)
            iv = 0x5F3759DF - (iv >> 1)
            y = plsc.bitcast(iv, jnp.float32)
            half_v = 0.5 * v
            for _n in range(3):
                y = y * (1.5 - half_v * y * y)
            stats.append((mean_v, y))
        for j in range(NJ):
            sl = pl.ds(j * 16, 16)
            g = g_v[sl]
            b = b_v[sl]
            for k in range(TUNROLL):
                t = t0 + k
                mean_v, y = stats[k]
                x = rows_v[t, sl]
                rows_v[t, sl] = (x - mean_v) * y * g + b
        return 0

    lax.fori_loop(0, G // TUNROLL, token_body, 0)


def _sc_body(ids_hbm, wt_hbm, pt_hbm, g_hbm, b_hbm, out_hbm,
             idx_v, rows_v, pos_v, g_v, b_v, sem):
    wid = lax.axis_index("s") * 2 + lax.axis_index("c")
    p0 = wid * P_PER_W
    pltpu.sync_copy(g_hbm, g_v)
    pltpu.sync_copy(b_hbm, b_v)

    def pc_body(pc, _):
        pbase = p0 + pc * G
        pltpu.sync_copy(pt_hbm.at[pl.ds(pbase, G)], pos_v)

        def b_body(bb, _):
            tok = bb * S + pbase
            pltpu.sync_copy(ids_hbm.at[pl.ds(tok, G)], idx_v)
            pltpu.async_copy(wt_hbm.at[idx_v], rows_v, sem).wait()
            _ln_chunk(rows_v, pos_v, g_v, b_v)
            pltpu.sync_copy(rows_v, out_hbm.at[pl.ds(tok, G)])
            return 0

        lax.fori_loop(0, B, b_body, 0)
        return 0

    lax.fori_loop(0, P_PER_W // G, pc_body, 0)


@jax.jit
def _run(ids_flat, word_table, pos_table, gamma, beta):
    mesh = plsc.VectorSubcoreMesh(core_axis_name="c", subcore_axis_name="s")
    k = pl.kernel(
        _sc_body,
        out_type=jax.ShapeDtypeStruct((B * S, D_MODEL), jnp.float32),
        mesh=mesh,
        compiler_params=pltpu.CompilerParams(needs_layout_passes=False),
        scratch_types=[
            pltpu.VMEM((G,), jnp.int32),
            pltpu.VMEM((G, D_MODEL), jnp.float32),
            pltpu.VMEM((G, D_MODEL), jnp.float32),
            pltpu.VMEM((D_MODEL,), jnp.float32),
            pltpu.VMEM((D_MODEL,), jnp.float32),
            pltpu.SemaphoreType.DMA,
        ],
    )
    return k(ids_flat, word_table, pos_table, gamma, beta)


def kernel(input_ids, word_table, pos_table, gamma, beta):
    ids_flat = jnp.reshape(input_ids.astype(jnp.int32), (B * S,))
    out = _run(ids_flat, word_table, pos_table, gamma, beta)
    return jnp.reshape(out, (B, S, D_MODEL))


# speedup vs baseline: 1.4153x; 1.2000x over previous
import jax
import jax.numpy as jnp
from jax import lax
from jax.experimental import pallas as pl
from jax.experimental.pallas import tpu as pltpu
from jax.experimental.pallas import tpu_sc as plsc

D_MODEL = 768
B = 4
S = 4096
EPS = 1e-12
NW = 32
P_PER_W = S // NW
G = 64
NJ = D_MODEL // 16
TUNROLL = 4


def _ln_chunk(rows_v, pos_v, g_v, b_v):
    @plsc.parallel_loop(0, G, 1, unroll=TUNROLL)
    def token_body(t):
        acc = jnp.zeros((16,), jnp.float32)
        acc2 = jnp.zeros((16,), jnp.float32)
        for j in range(NJ):
            sl = pl.ds(j * 16, 16)
            x = rows_v[t, sl] + pos_v[t, sl]
            rows_v[t, sl] = x
            acc = acc + x
            acc2 = acc2 + x * x
        mean = jnp.sum(acc) * (1.0 / D_MODEL)
        var = jnp.sum(acc2) * (1.0 / D_MODEL) - mean * mean
        ones = jnp.ones((16,), jnp.float32)
        mean_v = mean * ones
        v = (var + EPS) * ones
        iv = plsc.bitcast(v, jnp.int32)
        iv = 0x5F3759DF - (iv >> 1)
        y = plsc.bitcast(iv, jnp.float32)
        half_v = 0.5 * v
        for _n in range(3):
            y = y * (1.5 - half_v * y * y)
        for j in range(NJ):
            sl = pl.ds(j * 16, 16)
            x = rows_v[t, sl]
            rows_v[t, sl] = (x - mean_v) * y * g_v[sl] + b_v[sl]


def _sc_body(ids_hbm, wt_hbm, pt_hbm, g_hbm, b_hbm, out_hbm,
             idx_v, rows_v, pos_v, g_v, b_v, sem):
    wid = lax.axis_index("s") * 2 + lax.axis_index("c")
    p0 = wid * P_PER_W
    pltpu.sync_copy(g_hbm, g_v)
    pltpu.sync_copy(b_hbm, b_v)

    def pc_body(pc, _):
        pbase = p0 + pc * G
        pltpu.sync_copy(pt_hbm.at[pl.ds(pbase, G)], pos_v)

        def b_body(bb, _):
            tok = bb * S + pbase
            pltpu.sync_copy(ids_hbm.at[pl.ds(tok, G)], idx_v)
            pltpu.async_copy(wt_hbm.at[idx_v], rows_v, sem).wait()
            _ln_chunk(rows_v, pos_v, g_v, b_v)
            pltpu.sync_copy(rows_v, out_hbm.at[pl.ds(tok, G)])
            return 0

        lax.fori_loop(0, B, b_body, 0)
        return 0

    lax.fori_loop(0, P_PER_W // G, pc_body, 0)


@jax.jit
def _run(ids_flat, word_table, pos_table, gamma, beta):
    mesh = plsc.VectorSubcoreMesh(core_axis_name="c", subcore_axis_name="s")
    k = pl.kernel(
        _sc_body,
        out_type=jax.ShapeDtypeStruct((B * S, D_MODEL), jnp.float32),
        mesh=mesh,
        compiler_params=pltpu.CompilerParams(needs_layout_passes=False),
        scratch_types=[
            pltpu.VMEM((G,), jnp.int32),
            pltpu.VMEM((G, D_MODEL), jnp.float32),
            pltpu.VMEM((G, D_MODEL), jnp.float32),
            pltpu.VMEM((D_MODEL,), jnp.float32),
            pltpu.VMEM((D_MODEL,), jnp.float32),
            pltpu.SemaphoreType.DMA,
        ],
    )
    return k(ids_flat, word_table, pos_table, gamma, beta)


def kernel(input_ids, word_table, pos_table, gamma, beta):
    ids_flat = jnp.reshape(input_ids.astype(jnp.int32), (B * S,))
    out = _run(ids_flat, word_table, pos_table, gamma, beta)
    return jnp.reshape(out, (B, S, D_MODEL))


# speedup vs baseline: 2.6561x; 1.8767x over previous
import jax
import jax.numpy as jnp
from jax import lax
from jax.experimental import pallas as pl
from jax.experimental.pallas import tpu as pltpu
from jax.experimental.pallas import tpu_sc as plsc

D_MODEL = 768
B = 4
S = 4096
EPS = 1e-12
NW = 32
P_PER_W = S // NW
G = 64
NJ = D_MODEL // 16
TUNROLL = 2


def _ln_chunk(rows_v, pos_v, g_v, b_v):
    @plsc.parallel_loop(0, G, 1, unroll=TUNROLL)
    def token_body(t):
        acc = jnp.zeros((16,), jnp.float32)
        acc2 = jnp.zeros((16,), jnp.float32)
        for j in range(NJ):
            sl = pl.ds(j * 16, 16)
            x = rows_v[t, sl] + pos_v[t, sl]
            rows_v[t, sl] = x
            acc = acc + x
            acc2 = acc2 + x * x
        mean = jnp.sum(acc) * (1.0 / D_MODEL)
        var = jnp.sum(acc2) * (1.0 / D_MODEL) - mean * mean
        ones = jnp.ones((16,), jnp.float32)
        mean_v = mean * ones
        v = (var + EPS) * ones
        iv = plsc.bitcast(v, jnp.int32)
        iv = 0x5F3759DF - (iv >> 1)
        y = plsc.bitcast(iv, jnp.float32)
        half_v = 0.5 * v
        for _n in range(3):
            y = y * (1.5 - half_v * y * y)
        for j in range(NJ):
            sl = pl.ds(j * 16, 16)
            x = rows_v[t, sl]
            rows_v[t, sl] = (x - mean_v) * y * g_v[sl] + b_v[sl]


def _sc_body(ids_hbm, wt_hbm, pt_hbm, g_hbm, b_hbm, out_hbm,
             idx_v, rows_v, pos_v, g_v, b_v, sem):
    wid = lax.axis_index("s") * 2 + lax.axis_index("c")
    p0 = wid * P_PER_W
    pltpu.sync_copy(g_hbm, g_v)
    pltpu.sync_copy(b_hbm, b_v)

    def pc_body(pc, _):
        pbase = p0 + pc * G
        pltpu.sync_copy(pt_hbm.at[pl.ds(pbase, G)], pos_v)

        def b_body(bb, _):
            tok = bb * S + pbase
            pltpu.sync_copy(ids_hbm.at[pl.ds(tok, G)], idx_v)
            pltpu.async_copy(wt_hbm.at[idx_v], rows_v, sem).wait()
            _ln_chunk(rows_v, pos_v, g_v, b_v)
            pltpu.sync_copy(rows_v, out_hbm.at[pl.ds(tok, G)])
            return 0

        lax.fori_loop(0, B, b_body, 0)
        return 0

    lax.fori_loop(0, P_PER_W // G, pc_body, 0)


@jax.jit
def _run(ids_flat, word_table, pos_table, gamma, beta):
    mesh = plsc.VectorSubcoreMesh(core_axis_name="c", subcore_axis_name="s")
    k = pl.kernel(
        _sc_body,
        out_type=jax.ShapeDtypeStruct((B * S, D_MODEL), jnp.float32),
        mesh=mesh,
        compiler_params=pltpu.CompilerParams(needs_layout_passes=False),
        scratch_types=[
            pltpu.VMEM((G,), jnp.int32),
            pltpu.VMEM((G, D_MODEL), jnp.float32),
            pltpu.VMEM((G, D_MODEL), jnp.float32),
            pltpu.VMEM((D_MODEL,), jnp.float32),
            pltpu.VMEM((D_MODEL,), jnp.float32),
            pltpu.SemaphoreType.DMA,
        ],
    )
    return k(ids_flat, word_table, pos_table, gamma, beta)


def kernel(input_ids, word_table, pos_table, gamma, beta):
    ids_flat = jnp.reshape(input_ids.astype(jnp.int32), (B * S,))
    out = _run(ids_flat, word_table, pos_table, gamma, beta)
    return jnp.reshape(out, (B, S, D_MODEL))


# speedup vs baseline: 3.6307x; 1.3669x over previous
import jax
import jax.numpy as jnp
from jax import lax
from jax.experimental import pallas as pl
from jax.experimental.pallas import tpu as pltpu
from jax.experimental.pallas import tpu_sc as plsc

D_MODEL = 768
B = 4
S = 4096
EPS = 1e-12
NW = 32
P_PER_W = S // NW
G = 64
NJ = D_MODEL // 16
TUNROLL = 1


def _ln_chunk(rows_v, pos_v, g_v, b_v):
    @plsc.parallel_loop(0, G, 1, unroll=TUNROLL)
    def token_body(t):
        acc = jnp.zeros((16,), jnp.float32)
        acc2 = jnp.zeros((16,), jnp.float32)
        for j in range(NJ):
            sl = pl.ds(j * 16, 16)
            x = rows_v[t, sl] + pos_v[t, sl]
            rows_v[t, sl] = x
            acc = acc + x
            acc2 = acc2 + x * x
        mean = jnp.sum(acc) * (1.0 / D_MODEL)
        var = jnp.sum(acc2) * (1.0 / D_MODEL) - mean * mean
        ones = jnp.ones((16,), jnp.float32)
        mean_v = mean * ones
        v = (var + EPS) * ones
        iv = plsc.bitcast(v, jnp.int32)
        iv = 0x5F3759DF - (iv >> 1)
        y = plsc.bitcast(iv, jnp.float32)
        half_v = 0.5 * v
        for _n in range(3):
            y = y * (1.5 - half_v * y * y)
        for j in range(NJ):
            sl = pl.ds(j * 16, 16)
            x = rows_v[t, sl]
            rows_v[t, sl] = (x - mean_v) * y * g_v[sl] + b_v[sl]


def _sc_body(ids_hbm, wt_hbm, pt_hbm, g_hbm, b_hbm, out_hbm,
             idx_v, rows_v, pos_v, g_v, b_v, sem):
    wid = lax.axis_index("s") * 2 + lax.axis_index("c")
    p0 = wid * P_PER_W
    pltpu.sync_copy(g_hbm, g_v)
    pltpu.sync_copy(b_hbm, b_v)

    def pc_body(pc, _):
        pbase = p0 + pc * G
        pltpu.sync_copy(pt_hbm.at[pl.ds(pbase, G)], pos_v)

        def b_body(bb, _):
            tok = bb * S + pbase
            pltpu.sync_copy(ids_hbm.at[pl.ds(tok, G)], idx_v)
            pltpu.async_copy(wt_hbm.at[idx_v], rows_v, sem).wait()
            _ln_chunk(rows_v, pos_v, g_v, b_v)
            pltpu.sync_copy(rows_v, out_hbm.at[pl.ds(tok, G)])
            return 0

        lax.fori_loop(0, B, b_body, 0)
        return 0

    lax.fori_loop(0, P_PER_W // G, pc_body, 0)


@jax.jit
def _run(ids_flat, word_table, pos_table, gamma, beta):
    mesh = plsc.VectorSubcoreMesh(core_axis_name="c", subcore_axis_name="s")
    k = pl.kernel(
        _sc_body,
        out_type=jax.ShapeDtypeStruct((B * S, D_MODEL), jnp.float32),
        mesh=mesh,
        compiler_params=pltpu.CompilerParams(needs_layout_passes=False),
        scratch_types=[
            pltpu.VMEM((G,), jnp.int32),
            pltpu.VMEM((G, D_MODEL), jnp.float32),
            pltpu.VMEM((G, D_MODEL), jnp.float32),
            pltpu.VMEM((D_MODEL,), jnp.float32),
            pltpu.VMEM((D_MODEL,), jnp.float32),
            pltpu.SemaphoreType.DMA,
        ],
    )
    return k(ids_flat, word_table, pos_table, gamma, beta)


def kernel(input_ids, word_table, pos_table, gamma, beta):
    ids_flat = jnp.reshape(input_ids.astype(jnp.int32), (B * S,))
    out = _run(ids_flat, word_table, pos_table, gamma, beta)
    return jnp.reshape(out, (B, S, D_MODEL))
